# Initial kernel scaffold; baseline (speedup 1.0000x reference)
import functools

import jax
import jax.numpy as jnp
from jax import lax
from jax.experimental import pallas as pl
from jax.experimental.pallas import tpu as pltpu

D = 1024
E = 8
K = 2
D_FF = 256
N = 4096

T_BLK = 256
NEG_INF = -1e30


def _router_body(x_ref, rw_ref, w_ref):
    x = x_ref[...]
    rw = rw_ref[...]
    logits = lax.dot_general(x, rw, (((1,), (1,)), ((), ())),
                             preferred_element_type=jnp.float32)
    e_iota = lax.broadcasted_iota(jnp.int32, logits.shape, 1)
    m1 = jnp.max(logits, axis=1, keepdims=True)
    i1 = jnp.min(jnp.where(logits == m1, e_iota, E), axis=1, keepdims=True)
    masked = jnp.where(e_iota == i1, NEG_INF, logits)
    m2 = jnp.max(masked, axis=1, keepdims=True)
    i2 = jnp.min(jnp.where(masked == m2, e_iota, E), axis=1, keepdims=True)
    t = jnp.exp(m2 - m1)
    w2 = t / (1.0 + t)
    w1 = 1.0 - w2
    w_ref[...] = (jnp.where(e_iota == i1, w1, 0.0)
                  + jnp.where(e_iota == i2, w2, 0.0))


def _router(x, route_W):
    return pl.pallas_call(
        _router_body,
        grid=(N // T_BLK,),
        in_specs=[
            pl.BlockSpec((T_BLK, D), lambda t: (t, 0)),
            pl.BlockSpec((E, D), lambda t: (0, 0)),
        ],
        out_specs=pl.BlockSpec((T_BLK, E), lambda t: (t, 0)),
        out_shape=jax.ShapeDtypeStruct((N, E), jnp.float32),
    )(x, route_W)


def _dense_body(x_ref, w1_ref, b1_ref, w2_ref, b2_ref, fw_ref, out_ref):
    e = pl.program_id(1)
    x = x_ref[...]
    h = lax.dot_general(x, w1_ref[0], (((1,), (1,)), ((), ())),
                        preferred_element_type=jnp.float32)
    h = jnp.maximum(h + b1_ref[...], 0.0)
    y = lax.dot_general(h, w2_ref[0], (((1,), (1,)), ((), ())),
                        preferred_element_type=jnp.float32)
    y = jnp.maximum(y + b2_ref[...], 0.0)
    w = fw_ref[...][:, e][:, None]
    contrib = w * y

    @pl.when(e == 0)
    def _():
        out_ref[...] = contrib

    @pl.when(e != 0)
    def _():
        out_ref[...] += contrib


def _dense_experts(x, W1, b1, W2, b2, full_w):
    return pl.pallas_call(
        _dense_body,
        grid=(N // T_BLK, E),
        in_specs=[
            pl.BlockSpec((T_BLK, D), lambda t, e: (t, 0)),
            pl.BlockSpec((1, D_FF, D), lambda t, e: (e, 0, 0)),
            pl.BlockSpec((1, D_FF), lambda t, e: (e, 0)),
            pl.BlockSpec((1, D, D_FF), lambda t, e: (e, 0, 0)),
            pl.BlockSpec((1, D), lambda t, e: (e, 0)),
            pl.BlockSpec((T_BLK, E), lambda t, e: (t, 0)),
        ],
        out_specs=pl.BlockSpec((T_BLK, D), lambda t, e: (t, 0)),
        out_shape=jax.ShapeDtypeStruct((N, D), jnp.float32),
    )(x, W1, b1, W2, b2, full_w)


def kernel(x, route_W, W1, b1, W2, b2):
    full_w = _router(x, route_W)
    return _dense_experts(x, W1, b1, W2, b2, full_w)


# speedup vs baseline: 1.3279x; 1.3279x over previous
import functools

import jax
import jax.numpy as jnp
from jax import lax
from jax.experimental import pallas as pl
from jax.experimental.pallas import tpu as pltpu

D = 1024
E = 8
K = 2
D_FF = 256
N = 4096

T_BLK = 256
NEG_INF = -1e30


def _router_body(x_ref, rw_ref, w_ref):
    x = x_ref[...]
    rw = rw_ref[...]
    logits = lax.dot_general(x, rw, (((1,), (1,)), ((), ())),
                             preferred_element_type=jnp.float32)
    e_iota = lax.broadcasted_iota(jnp.int32, logits.shape, 1)
    m1 = jnp.max(logits, axis=1, keepdims=True)
    i1 = jnp.min(jnp.where(logits == m1, e_iota, E), axis=1, keepdims=True)
    masked = jnp.where(e_iota == i1, NEG_INF, logits)
    m2 = jnp.max(masked, axis=1, keepdims=True)
    i2 = jnp.min(jnp.where(masked == m2, e_iota, E), axis=1, keepdims=True)
    t = jnp.exp(m2 - m1)
    w2 = t / (1.0 + t)
    w1 = 1.0 - w2
    w_ref[...] = (jnp.where(e_iota == i1, w1, 0.0)
                  + jnp.where(e_iota == i2, w2, 0.0))


def _router(x, route_W):
    return pl.pallas_call(
        _router_body,
        grid=(N // T_BLK,),
        in_specs=[
            pl.BlockSpec((T_BLK, D), lambda t: (t, 0)),
            pl.BlockSpec((E, D), lambda t: (0, 0)),
        ],
        out_specs=pl.BlockSpec((T_BLK, E), lambda t: (t, 0)),
        out_shape=jax.ShapeDtypeStruct((N, E), jnp.float32),
    )(x, route_W)


def _dense_body(x_ref, w1_ref, b1_ref, w2_ref, b2_ref, fw_ref, out_ref):
    e = pl.program_id(1)
    x = x_ref[...]
    h = lax.dot_general(x, w1_ref[0], (((1,), (1,)), ((), ())),
                        preferred_element_type=jnp.float32)
    h = jnp.maximum(h + b1_ref[0], 0.0)
    y = lax.dot_general(h, w2_ref[0], (((1,), (1,)), ((), ())),
                        preferred_element_type=jnp.float32)
    y = jnp.maximum(y + b2_ref[0], 0.0)
    fw = fw_ref[...]
    lane = lax.broadcasted_iota(jnp.int32, fw.shape, 1)
    w = jnp.sum(jnp.where(lane == e, fw, 0.0), axis=1, keepdims=True)
    contrib = w * y

    @pl.when(e == 0)
    def _():
        out_ref[...] = contrib

    @pl.when(e != 0)
    def _():
        out_ref[...] += contrib


def _dense_experts(x, W1, b1, W2, b2, full_w):
    return pl.pallas_call(
        _dense_body,
        grid=(N // T_BLK, E),
        in_specs=[
            pl.BlockSpec((T_BLK, D), lambda t, e: (t, 0)),
            pl.BlockSpec((1, D_FF, D), lambda t, e: (e, 0, 0)),
            pl.BlockSpec((1, 1, D_FF), lambda t, e: (e, 0, 0)),
            pl.BlockSpec((1, D, D_FF), lambda t, e: (e, 0, 0)),
            pl.BlockSpec((1, 1, D), lambda t, e: (e, 0, 0)),
            pl.BlockSpec((T_BLK, E), lambda t, e: (t, 0)),
        ],
        out_specs=pl.BlockSpec((T_BLK, D), lambda t, e: (t, 0)),
        out_shape=jax.ShapeDtypeStruct((N, D), jnp.float32),
    )(x, W1, b1.reshape(E, 1, D_FF), W2, b2.reshape(E, 1, D), full_w)


def kernel(x, route_W, W1, b1, W2, b2):
    full_w = _router(x, route_W)
    return _dense_experts(x, W1, b1, W2, b2, full_w)
